# baseline (speedup 1.0000x reference)
import functools

import jax
import jax.numpy as jnp
from jax import lax
from jax.experimental import pallas as pl
from jax.experimental.pallas import tpu as pltpu
from jax.experimental.pallas import tpu_sc as plsc

B = 64
S = 8
V = 100000
NC = 2
NS = 16
L = 16
RPW = 2

CH = 12800
NCH = 6
SC_COLS = NCH * CH
MAIN = 99840
TAILN = V - MAIN
TAILP = 256
SUB = 1280
NSUBID = MAIN // SUB
UNROLL = 16
SUBVECS = SUB // L

NEG_INF = float("-inf")
I32_BIG = 2**31 - 1


def _scan_region(read_vec, nvec, gmax, gchunk, cid, unroll=UNROLL):
    iters = nvec // unroll

    def inner(i, accs):
        return tuple(
            jnp.maximum(a, read_vec(i * unroll + r)) for r, a in enumerate(accs)
        )

    accs = tuple(read_vec(r) for r in range(unroll))
    if iters > 1:
        accs = lax.fori_loop(1, iters, inner, accs)
    macc = accs[0]
    for a in accs[1:]:
        macc = jnp.maximum(macc, a)

    cid_v = jnp.full((L,), cid, dtype=jnp.int32)
    better = macc > gmax
    gmax = jnp.where(better, macc, gmax)
    gchunk = jnp.where(better, cid_v, gchunk)
    return gmax, gchunk


def _argmax_kernel(x_hbm, tail_hbm, out_hbm, buf_a, buf_b, buf_c, rbuf, t160,
                   res_v, idx2, idx2t, idx_a, idx_b, sem0, sem1, sem2, semt):
    c = lax.axis_index("c")
    s = lax.axis_index("s")
    w = NS * c + s
    b0 = RPW * w
    sems = (sem0, sem1, sem2)
    bufs = (buf_a, buf_b, buf_c)
    idx1s = (idx_a, idx_b)

    lane = lax.iota(jnp.int32, L)
    plsc.store_scatter(idx2, [lane], 8 * (b0 + lane) + 7, mask=lane < RPW)
    plsc.store_scatter(idx_a, [lane], jnp.full((L,), 8 * b0 + 7, jnp.int32),
                       mask=lane < 1)
    plsc.store_scatter(idx_b, [lane], jnp.full((L,), 8 * b0 + 15, jnp.int32),
                       mask=lane < 1)

    plsc.store_scatter(idx2t, [lane], b0 + lane, mask=lane < RPW)
    tdesc = pltpu.make_async_copy(tail_hbm.at[idx2t], t160, semt)
    tdesc.start()

    chunks = [(k * CH, CH, k * (CH // SUB)) for k in range(NCH)]

    def chunk_copy(k, p):
        off, width, _ = chunks[k]
        dst = bufs[p] if width == CH else bufs[p].at[:, pl.ds(0, width)]
        return pltpu.make_async_copy(
            x_hbm.at[idx2, pl.ds(off, width)], dst, sems[p])

    NBUF = len(bufs)
    descs = [None] * NBUF
    for k0 in range(NBUF - 1):
        descs[k0] = chunk_copy(k0, k0)
        descs[k0].start()

    gmax = [jnp.full((L,), NEG_INF, jnp.float32) for _ in range(RPW)]
    gchunk = [jnp.full((L,), I32_BIG, jnp.int32) for _ in range(RPW)]

    for k in range(len(chunks)):
        p = k % NBUF
        if k + NBUF - 1 < len(chunks):
            q = (k + NBUF - 1) % NBUF
            descs[q] = chunk_copy(k + NBUF - 1, q)
            descs[q].start()
        descs[p].wait()
        _, width, cid0 = chunks[k]
        nsub = width // SUB
        buf = bufs[p]
        for j in range(RPW):

            def sub_body(t, carry, _j=j, _buf=buf, _cid0=cid0):
                gm, gc = carry
                base = t * SUB
                return _scan_region(
                    lambda i: _buf[_j, pl.ds(base + i * L, L)],
                    SUBVECS, gm, gc, _cid0 + t)

            gmax[j], gchunk[j] = lax.fori_loop(
                0, nsub, sub_body, (gmax[j], gchunk[j]))

    tdesc.wait()
    for j in range(RPW):
        gmax[j], gchunk[j] = _scan_region(
            lambda i, _j=j: t160[_j, pl.ds(i * L, L)],
            TAILP // L, gmax[j], gchunk[j], NSUBID, unroll=4)

    res = jnp.zeros((L,), jnp.int32)
    for j in range(RPW):
        m = jnp.max(gmax[j])
        mvec = jnp.full((L,), m)
        cand = jnp.where(gmax[j] == mvec, gchunk[j],
                         jnp.full((L,), I32_BIG, jnp.int32))
        cstar = jnp.min(cand)

        @pl.when(cstar < NSUBID)
        def _(j=j, cstar=cstar):
            pltpu.sync_copy(x_hbm.at[idx1s[j], pl.ds(cstar * SUB, SUB)], rbuf)

        big = jnp.full((L,), I32_BIG, jnp.int32)

        def match_min(read_vec, nvec):
            def body(i, fidx):
                v = read_vec(i)
                idx = lane + jnp.full((L,), i * L, dtype=jnp.int32)
                return jnp.minimum(fidx, jnp.where(v == mvec, idx, big))
            return lax.fori_loop(0, nvec, body, big)

        fidx = lax.cond(
            cstar < NSUBID,
            lambda: match_min(lambda i: rbuf[0, pl.ds(i * L, L)], SUBVECS),
            lambda j=j: match_min(lambda i: t160[j, pl.ds(i * L, L)],
                                  TAILP // L),
        )
        ans = jnp.min(fidx) + cstar * SUB
        res = jnp.where(lane == jnp.full((L,), j, jnp.int32),
                        jnp.full((L,), ans), res)
        res = jnp.where(lane == jnp.full((L,), RPW + j, jnp.int32),
                        plsc.bitcast(mvec, jnp.int32), res)

    res_v[...] = res
    pltpu.sync_copy(res_v, out_hbm.at[w])


@functools.partial(
    pl.kernel,
    out_type=jax.ShapeDtypeStruct((B // RPW, L), jnp.int32),
    mesh=plsc.VectorSubcoreMesh(core_axis_name="c", subcore_axis_name="s"),
    scratch_types=[
        pltpu.VMEM((RPW, CH), jnp.float32),
        pltpu.VMEM((RPW, CH), jnp.float32),
        pltpu.VMEM((RPW, CH), jnp.float32),
        pltpu.VMEM((1, SUB), jnp.float32),
        pltpu.VMEM((RPW, TAILP), jnp.float32),
        pltpu.VMEM((L,), jnp.int32),
        pltpu.VMEM((RPW,), jnp.int32),
        pltpu.VMEM((RPW,), jnp.int32),
        pltpu.VMEM((1,), jnp.int32),
        pltpu.VMEM((1,), jnp.int32),
        pltpu.SemaphoreType.DMA,
        pltpu.SemaphoreType.DMA,
        pltpu.SemaphoreType.DMA,
        pltpu.SemaphoreType.DMA,
    ],
    compiler_params=pltpu.CompilerParams(needs_layout_passes=False),
)
def _sc_argmax(x_hbm, tail_hbm, out_hbm, *scratch):
    _argmax_kernel(x_hbm, tail_hbm, out_hbm, *scratch)


TCW = 7680
NTCB = (MAIN - SC_COLS) // TCW


def _tc_body(x_ref, m_ref, i_ref):
    i = pl.program_id(0)
    k = pl.program_id(1)
    row = x_ref[0, S - 1 : S, :]
    m_k = jnp.max(row)
    i_k = jnp.argmax(row).astype(jnp.int32) + SC_COLS + k * TCW
    m_row = jnp.full((1, 128), m_k, jnp.float32)
    i_row = jnp.full((1, 128), i_k, jnp.int32)

    @pl.when(k == 0)
    def _():
        m_ref[pl.ds(i, 1), :] = m_row
        i_ref[pl.ds(i, 1), :] = i_row

    @pl.when(k > 0)
    def _():
        prev_m = m_ref[pl.ds(i, 1), :]
        prev_i = i_ref[pl.ds(i, 1), :]
        better = m_row > prev_m
        m_ref[pl.ds(i, 1), :] = jnp.where(better, m_row, prev_m)
        i_ref[pl.ds(i, 1), :] = jnp.where(better, i_row, prev_i)


_tc_argmax = pl.pallas_call(
    _tc_body,
    grid=(B, NTCB),
    in_specs=[pl.BlockSpec((1, S, TCW),
                           lambda i, k: (i, 0, SC_COLS // TCW + k))],
    out_specs=[pl.BlockSpec((B, 128), lambda i, k: (0, 0)),
               pl.BlockSpec((B, 128), lambda i, k: (0, 0))],
    out_shape=[jax.ShapeDtypeStruct((B, 128), jnp.float32),
               jax.ShapeDtypeStruct((B, 128), jnp.int32)],
)


def kernel(logits):
    x2d = logits.reshape(B * S, V)
    tail = logits[:, -1, MAIN:]
    tail = jnp.pad(tail, ((0, 0), (0, TAILP - TAILN)),
                   constant_values=-jnp.inf)
    out = _sc_argmax(x2d, tail)
    tc_m2, tc_idx2 = _tc_argmax(logits)
    tc_m, tc_idx = tc_m2[:, 0], tc_idx2[:, 0]
    sc_idx = out[:, :RPW].reshape(B)
    sc_m = jax.lax.bitcast_convert_type(out[:, RPW:2 * RPW],
                                        jnp.float32).reshape(B)
    tc_wins = (tc_m > sc_m) | ((tc_m == sc_m) & (tc_idx < sc_idx))
    return jnp.where(tc_wins, tc_idx, sc_idx).reshape(B, 1)


# speedup vs baseline: 3.5491x; 3.5491x over previous
import functools

import jax
import jax.numpy as jnp
from jax import lax
from jax.experimental import pallas as pl
from jax.experimental.pallas import tpu as pltpu
from jax.experimental.pallas import tpu_sc as plsc

B = 64
S = 8
V = 100000
NC = 2
NS = 16
L = 16
RPW = 2

CH = 12800
NCH = 7
LASTCH = 10240
MAIN = NCH * CH + LASTCH
TAILN = V - MAIN
TAILP = 256
SUB = 1280
NSUBID = MAIN // SUB
UNROLL = 16
SUBVECS = SUB // L

NEG_INF = float("-inf")
I32_BIG = 2**31 - 1


def _scan_region(read_vec, nvec, gmax, gchunk, cid, unroll=UNROLL):
    iters = nvec // unroll

    def inner(i, accs):
        return tuple(
            jnp.maximum(a, read_vec(i * unroll + r)) for r, a in enumerate(accs)
        )

    accs = tuple(read_vec(r) for r in range(unroll))
    if iters > 1:
        accs = lax.fori_loop(1, iters, inner, accs)
    macc = accs[0]
    for a in accs[1:]:
        macc = jnp.maximum(macc, a)

    cid_v = jnp.full((L,), cid, dtype=jnp.int32)
    better = macc > gmax
    gmax = jnp.where(better, macc, gmax)
    gchunk = jnp.where(better, cid_v, gchunk)
    return gmax, gchunk


def _argmax_kernel(x_hbm, tail_hbm, out_hbm, buf_a, buf_b, buf_c, rbuf, t160,
                   res_v, idx2, idx2t, idx_a, idx_b, sem0, sem1, sem2, semt):
    c = lax.axis_index("c")
    s = lax.axis_index("s")
    w = NS * c + s
    b0 = RPW * w
    sems = (sem0, sem1, sem2)
    bufs = (buf_a, buf_b, buf_c)
    idx1s = (idx_a, idx_b)

    lane = lax.iota(jnp.int32, L)
    plsc.store_scatter(idx2, [lane], 8 * (b0 + lane) + 7, mask=lane < RPW)
    plsc.store_scatter(idx_a, [lane], jnp.full((L,), 8 * b0 + 7, jnp.int32),
                       mask=lane < 1)
    plsc.store_scatter(idx_b, [lane], jnp.full((L,), 8 * b0 + 15, jnp.int32),
                       mask=lane < 1)

    plsc.store_scatter(idx2t, [lane], b0 + lane, mask=lane < RPW)
    tdesc = pltpu.make_async_copy(tail_hbm.at[idx2t], t160, semt)
    tdesc.start()

    chunks = [(k * CH, CH, k * (CH // SUB)) for k in range(NCH)]
    chunks.append((NCH * CH, LASTCH, NCH * (CH // SUB)))

    def chunk_copy(k, p):
        off, width, _ = chunks[k]
        dst = bufs[p] if width == CH else bufs[p].at[:, pl.ds(0, width)]
        return pltpu.make_async_copy(
            x_hbm.at[idx2, pl.ds(off, width)], dst, sems[p])

    NBUF = len(bufs)
    descs = [None] * NBUF
    for k0 in range(NBUF - 1):
        descs[k0] = chunk_copy(k0, k0)
        descs[k0].start()

    gmax = [jnp.full((L,), NEG_INF, jnp.float32) for _ in range(RPW)]
    gchunk = [jnp.full((L,), I32_BIG, jnp.int32) for _ in range(RPW)]

    for k in range(len(chunks)):
        p = k % NBUF
        if k + NBUF - 1 < len(chunks):
            q = (k + NBUF - 1) % NBUF
            descs[q] = chunk_copy(k + NBUF - 1, q)
            descs[q].start()
        descs[p].wait()
        _, width, cid0 = chunks[k]
        nsub = width // SUB
        buf = bufs[p]
        for j in range(RPW):

            def sub_body(t, carry, _j=j, _buf=buf, _cid0=cid0):
                gm, gc = carry
                base = t * SUB
                return _scan_region(
                    lambda i: _buf[_j, pl.ds(base + i * L, L)],
                    SUBVECS, gm, gc, _cid0 + t)

            gmax[j], gchunk[j] = lax.fori_loop(
                0, nsub, sub_body, (gmax[j], gchunk[j]))

    tdesc.wait()
    for j in range(RPW):
        gmax[j], gchunk[j] = _scan_region(
            lambda i, _j=j: t160[_j, pl.ds(i * L, L)],
            TAILP // L, gmax[j], gchunk[j], NSUBID, unroll=4)

    res = jnp.zeros((L,), jnp.int32)
    for j in range(RPW):
        m = jnp.max(gmax[j])
        mvec = jnp.full((L,), m)
        cand = jnp.where(gmax[j] == mvec, gchunk[j],
                         jnp.full((L,), I32_BIG, jnp.int32))
        cstar = jnp.min(cand)

        @pl.when(cstar < NSUBID)
        def _(j=j, cstar=cstar):
            pltpu.sync_copy(x_hbm.at[idx1s[j], pl.ds(cstar * SUB, SUB)], rbuf)

        big = jnp.full((L,), I32_BIG, jnp.int32)

        def match_min(read_vec, nvec):
            def body(i, fidx):
                v = read_vec(i)
                idx = lane + jnp.full((L,), i * L, dtype=jnp.int32)
                return jnp.minimum(fidx, jnp.where(v == mvec, idx, big))
            return lax.fori_loop(0, nvec, body, big)

        fidx = lax.cond(
            cstar < NSUBID,
            lambda: match_min(lambda i: rbuf[0, pl.ds(i * L, L)], SUBVECS),
            lambda j=j: match_min(lambda i: t160[j, pl.ds(i * L, L)],
                                  TAILP // L),
        )
        ans = jnp.min(fidx) + cstar * SUB
        res = jnp.where(lane == jnp.full((L,), j, jnp.int32),
                        jnp.full((L,), ans), res)

    res_v[...] = res
    pltpu.sync_copy(res_v, out_hbm.at[w])


@functools.partial(
    pl.kernel,
    out_type=jax.ShapeDtypeStruct((B // RPW, L), jnp.int32),
    mesh=plsc.VectorSubcoreMesh(core_axis_name="c", subcore_axis_name="s"),
    scratch_types=[
        pltpu.VMEM((RPW, CH), jnp.float32),
        pltpu.VMEM((RPW, CH), jnp.float32),
        pltpu.VMEM((RPW, CH), jnp.float32),
        pltpu.VMEM((1, SUB), jnp.float32),
        pltpu.VMEM((RPW, TAILP), jnp.float32),
        pltpu.VMEM((L,), jnp.int32),
        pltpu.VMEM((RPW,), jnp.int32),
        pltpu.VMEM((RPW,), jnp.int32),
        pltpu.VMEM((1,), jnp.int32),
        pltpu.VMEM((1,), jnp.int32),
        pltpu.SemaphoreType.DMA,
        pltpu.SemaphoreType.DMA,
        pltpu.SemaphoreType.DMA,
        pltpu.SemaphoreType.DMA,
    ],
    compiler_params=pltpu.CompilerParams(needs_layout_passes=False),
)
def _sc_argmax(x_hbm, tail_hbm, out_hbm, *scratch):
    _argmax_kernel(x_hbm, tail_hbm, out_hbm, *scratch)


def kernel(logits):
    x2d = logits.reshape(B * S, V)
    tail = logits[:, -1, MAIN:]
    tail = jnp.pad(tail, ((0, 0), (0, TAILP - TAILN)),
                   constant_values=-jnp.inf)
    out = _sc_argmax(x2d, tail)
    return out[:, :RPW].reshape(B, 1)
